# baseline (speedup 1.0000x reference)
import functools

import jax
import jax.numpy as jnp
from jax.experimental import pallas as pl

T = 4096
D_MODEL = 1024
E = 16
TOPK = 2
BOTTLENECK = 64
SCALE = 0.5
EB = E * BOTTLENECK

TB = 512


def _moe_block_kernel(x_ref, wg_ref, wd_ref, bd_ref, wu_ref, bu_ref, out_ref):
    xb = x_ref[:]
    logits = jnp.dot(xb, wg_ref[:], preferred_element_type=jnp.float32)
    idx = jax.lax.broadcasted_iota(jnp.int32, logits.shape, 1)
    v1 = jnp.max(logits, axis=1, keepdims=True)
    i1 = jnp.min(jnp.where(logits == v1, idx, E), axis=1, keepdims=True)
    m1 = idx == i1
    logits2 = jnp.where(m1, -jnp.inf, logits)
    v2 = jnp.max(logits2, axis=1, keepdims=True)
    i2 = jnp.min(jnp.where(logits2 == v2, idx, E), axis=1, keepdims=True)
    e2 = jnp.exp(v2 - v1)
    denom = 1.0 + e2
    g1 = 1.0 / denom
    g2 = e2 / denom
    gates = jnp.where(m1, g1, 0.0) + jnp.where(idx == i2, g2, 0.0)
    gates = gates * SCALE

    h = jnp.dot(xb, wd_ref[:], preferred_element_type=jnp.float32)
    h = jnp.maximum(h + bd_ref[:], 0.0)
    rep_e = jax.lax.broadcasted_iota(jnp.int32, (E, EB), 0)
    rep_l = jax.lax.broadcasted_iota(jnp.int32, (E, EB), 1) // BOTTLENECK
    rep = (rep_e == rep_l).astype(jnp.float32)
    gates_rep = jnp.dot(gates, rep, preferred_element_type=jnp.float32)
    hg = (h * gates_rep).astype(jnp.bfloat16)
    out = jnp.dot(hg, wu_ref[:], preferred_element_type=jnp.float32)
    out = out + jnp.dot(gates, bu_ref[:], preferred_element_type=jnp.float32)
    out_ref[:] = out


@functools.partial(jax.jit, static_argnames=())
def kernel(x, w_gate, w_noise, down_w, down_b, up_w, up_b):
    del w_noise
    xb = x.astype(jnp.bfloat16)
    wgb = w_gate.astype(jnp.bfloat16)
    wd = down_w.transpose(1, 0, 2).reshape(D_MODEL, EB).astype(jnp.bfloat16)
    wu = up_w.reshape(EB, D_MODEL).astype(jnp.bfloat16)
    bd = down_b.reshape(1, EB)
    grid = (T // TB,)
    return pl.pallas_call(
        _moe_block_kernel,
        grid=grid,
        in_specs=[
            pl.BlockSpec((TB, D_MODEL), lambda i: (i, 0)),
            pl.BlockSpec((D_MODEL, E), lambda i: (0, 0)),
            pl.BlockSpec((D_MODEL, EB), lambda i: (0, 0)),
            pl.BlockSpec((1, EB), lambda i: (0, 0)),
            pl.BlockSpec((EB, D_MODEL), lambda i: (0, 0)),
            pl.BlockSpec((E, D_MODEL), lambda i: (0, 0)),
        ],
        out_specs=pl.BlockSpec((TB, D_MODEL), lambda i: (i, 0)),
        out_shape=jax.ShapeDtypeStruct((T, D_MODEL), jnp.float32),
    )(xb, wgb, wd, bd, wu, up_b)


# speedup vs baseline: 1.2147x; 1.2147x over previous
import functools

import jax
import jax.numpy as jnp
from jax.experimental import pallas as pl

T = 4096
D_MODEL = 1024
E = 16
TOPK = 2
BOTTLENECK = 64
SCALE = 0.5
EB = E * BOTTLENECK

TB = 512


def _moe_block_kernel(x_ref, wg_ref, wd_ref, bd_ref, wu_ref, bu_ref, out_ref):
    xb = x_ref[:].astype(jnp.bfloat16)
    logits = jnp.dot(xb, wg_ref[:], preferred_element_type=jnp.float32)
    idx = jax.lax.broadcasted_iota(jnp.int32, logits.shape, 1)
    v1 = jnp.max(logits, axis=1, keepdims=True)
    i1 = jnp.min(jnp.where(logits == v1, idx, E), axis=1, keepdims=True)
    m1 = idx == i1
    logits2 = jnp.where(m1, -jnp.inf, logits)
    v2 = jnp.max(logits2, axis=1, keepdims=True)
    i2 = jnp.min(jnp.where(logits2 == v2, idx, E), axis=1, keepdims=True)
    e2 = jnp.exp(v2 - v1)
    denom = 1.0 + e2
    g1 = 1.0 / denom
    g2 = e2 / denom
    gates = jnp.where(m1, g1, 0.0) + jnp.where(idx == i2, g2, 0.0)
    gates = gates * SCALE

    h = jnp.dot(xb, wd_ref[:], preferred_element_type=jnp.float32)
    h = jnp.maximum(h + bd_ref[:], 0.0)
    rep_e = jax.lax.broadcasted_iota(jnp.int32, (E, EB), 0)
    rep_l = jax.lax.broadcasted_iota(jnp.int32, (E, EB), 1) // BOTTLENECK
    rep = (rep_e == rep_l).astype(jnp.float32)
    gates_rep = jnp.dot(gates, rep, preferred_element_type=jnp.float32)
    hg = (h * gates_rep).astype(jnp.bfloat16)
    out = jnp.dot(hg, wu_ref[:], preferred_element_type=jnp.float32)
    out = out + jnp.dot(gates, bu_ref[:], preferred_element_type=jnp.float32)
    out_ref[:] = out


@functools.partial(jax.jit, static_argnames=())
def kernel(x, w_gate, w_noise, down_w, down_b, up_w, up_b):
    del w_noise
    wgb = w_gate.astype(jnp.bfloat16)
    wd = down_w.transpose(1, 0, 2).reshape(D_MODEL, EB).astype(jnp.bfloat16)
    wu = up_w.reshape(EB, D_MODEL).astype(jnp.bfloat16)
    bd = down_b.reshape(1, EB)
    grid = (T // TB,)
    return pl.pallas_call(
        _moe_block_kernel,
        grid=grid,
        in_specs=[
            pl.BlockSpec((TB, D_MODEL), lambda i: (i, 0)),
            pl.BlockSpec((D_MODEL, E), lambda i: (0, 0)),
            pl.BlockSpec((D_MODEL, EB), lambda i: (0, 0)),
            pl.BlockSpec((1, EB), lambda i: (0, 0)),
            pl.BlockSpec((EB, D_MODEL), lambda i: (0, 0)),
            pl.BlockSpec((E, D_MODEL), lambda i: (0, 0)),
        ],
        out_specs=pl.BlockSpec((TB, D_MODEL), lambda i: (i, 0)),
        out_shape=jax.ShapeDtypeStruct((T, D_MODEL), jnp.float32),
    )(x, wgb, wd, bd, wu, up_b)
